# baseline (speedup 1.0000x reference)
import jax
import jax.numpy as jnp
from jax import lax
from jax.experimental import pallas as pl
from jax.experimental.pallas import tpu as pltpu
from jax.experimental.pallas import tpu_sc as plsc

NDEV = 8
T = 4096
D = 1024
TOP_K = 2
NEXP = 16

NC = 2
NS = 16
NW = NC * NS
TPW = T // NW

TB = 256
NTB = T // TB


def _routing_body(idx_hbm, map_hbm, w_hbm, idx_v, map_v, wloc, wsem):
    cid = lax.axis_index("c")
    sid = lax.axis_index("s")
    wid = sid * NC + cid
    t0 = wid * TPW

    pltpu.sync_copy(idx_hbm.at[pl.ds(wid * (TPW * TOP_K), TPW * TOP_K)], idx_v)
    pltpu.sync_copy(map_hbm, map_v)

    def _zrow(i, carry):
        for j in range(TPW // 16):
            wloc[pl.ds(i * TPW + j * 16, 16)] = jnp.zeros((16,), jnp.float32)
        return carry
    lax.fori_loop(0, NDEV, _zrow, 0)

    io16 = lax.iota(jnp.int32, 16)
    ones = jnp.ones((16,), jnp.float32)
    for c in range(TPW // 16):
        lt = c * 16 + io16
        e0 = plsc.load_gather(idx_v, [2 * lt])
        e1 = plsc.load_gather(idx_v, [2 * lt + 1])
        d0 = plsc.load_gather(map_v, [e0])
        d1 = plsc.load_gather(map_v, [e1])
        plsc.store_scatter(wloc, [d0 * TPW + lt], ones)
        plsc.store_scatter(wloc, [d1 * TPW + lt], ones)

    hs = []
    for d in range(NDEV):
        hs.append(pltpu.async_copy(
            wloc.at[pl.ds(d * TPW, TPW)], w_hbm.at[d, pl.ds(t0, TPW)], wsem))
    for h in hs:
        h.wait()


def _routing_mask(idx_flat, expert_mapping):
    mesh = plsc.VectorSubcoreMesh(
        core_axis_name="c", subcore_axis_name="s",
        num_cores=NC, num_subcores=NS)
    f = pl.kernel(
        _routing_body,
        out_type=jax.ShapeDtypeStruct((NDEV, T), jnp.float32),
        mesh=mesh,
        compiler_params=pltpu.CompilerParams(needs_layout_passes=False),
        scratch_types=[
            pltpu.VMEM((TPW * TOP_K,), jnp.int32),
            pltpu.VMEM((NEXP,), jnp.int32),
            pltpu.VMEM((NDEV * TPW,), jnp.float32),
            pltpu.SemaphoreType.DMA,
        ],
    )
    return f(idx_flat, expert_mapping)


def _dispatch_tc(in_hbm, w_hbm, out_hbm, in_v, w_v, vbuf, ssems, osems, wsem):
    pltpu.make_async_copy(w_hbm, w_v, wsem).start()
    pltpu.make_async_copy(w_hbm, w_v, wsem).wait()

    def stage(tb, slot):
        return pltpu.make_async_copy(
            in_hbm.at[pl.ds(tb * TB, TB)], in_v.at[pl.ds(tb * TB, TB)],
            ssems.at[slot])

    stage(0, 0).start()
    stage(1, 1).start()

    def out_copy(tb, slot):
        return pltpu.make_async_copy(
            vbuf.at[slot], out_hbm.at[:, pl.ds(tb * TB, TB), :],
            osems.at[slot])

    def body(tb, carry):
        p = lax.rem(tb, 2)

        @pl.when(tb >= 2)
        def _():
            out_copy(tb, p).wait()

        stage(tb, p).wait()

        @pl.when(tb + 2 < NTB)
        def _():
            stage(tb + 2, p).start()

        rows = in_v[pl.ds(tb * TB, TB), :]
        wb = w_v[:, pl.ds(tb * TB, TB)]
        blk = rows[None, :, :] * wb[:, :, None]

        @pl.when(p == 0)
        def _():
            vbuf[0] = blk
            out_copy(tb, 0).start()

        @pl.when(p == 1)
        def _():
            vbuf[1] = blk
            out_copy(tb, 1).start()

        return carry

    lax.fori_loop(0, NTB, body, 0)
    out_copy(NTB - 2, 0).wait()
    out_copy(NTB - 1, 1).wait()


def kernel(input_tensor, expert_indices, expert_mapping):
    idx_flat = expert_indices.reshape(-1)
    w = _routing_mask(idx_flat, expert_mapping)
    out = pl.pallas_call(
        _dispatch_tc,
        in_specs=[
            pl.BlockSpec(memory_space=pl.ANY),
            pl.BlockSpec(memory_space=pl.ANY),
        ],
        out_specs=pl.BlockSpec(memory_space=pl.ANY),
        out_shape=jax.ShapeDtypeStruct((NDEV, T, D), jnp.float32),
        scratch_shapes=[
            pltpu.VMEM((T, D), jnp.float32),
            pltpu.VMEM((NDEV, T), jnp.float32),
            pltpu.VMEM((2, NDEV, TB, D), jnp.float32),
            pltpu.SemaphoreType.DMA((2,)),
            pltpu.SemaphoreType.DMA((2,)),
            pltpu.SemaphoreType.DMA,
        ],
    )(input_tensor, w)
    return out


# speedup vs baseline: 1.0206x; 1.0206x over previous
import jax
import jax.numpy as jnp
from jax import lax
from jax.experimental import pallas as pl
from jax.experimental.pallas import tpu as pltpu
from jax.experimental.pallas import tpu_sc as plsc

NDEV = 8
T = 4096
D = 1024
TOP_K = 2
NEXP = 16

NC = 2
NS = 16
NW = NC * NS
TPW = T // NW

TB = 256
NTB = T // TB


def _routing_body(idx_hbm, map_hbm, w_hbm, idx_v, map_v, wloc, wsem):
    cid = lax.axis_index("c")
    sid = lax.axis_index("s")
    wid = sid * NC + cid
    t0 = wid * TPW

    pltpu.sync_copy(idx_hbm.at[pl.ds(wid * (TPW * TOP_K), TPW * TOP_K)], idx_v)
    pltpu.sync_copy(map_hbm, map_v)

    def _zrow(i, carry):
        for j in range(TPW // 16):
            wloc[pl.ds(i * TPW + j * 16, 16)] = jnp.zeros((16,), jnp.float32)
        return carry
    lax.fori_loop(0, NDEV, _zrow, 0)

    io16 = lax.iota(jnp.int32, 16)
    ones = jnp.ones((16,), jnp.float32)
    for c in range(TPW // 16):
        lt = c * 16 + io16
        e0 = plsc.load_gather(idx_v, [2 * lt])
        e1 = plsc.load_gather(idx_v, [2 * lt + 1])
        d0 = plsc.load_gather(map_v, [e0])
        d1 = plsc.load_gather(map_v, [e1])
        plsc.store_scatter(wloc, [d0 * TPW + lt], ones)
        plsc.store_scatter(wloc, [d1 * TPW + lt], ones)

    hs = []
    for d in range(NDEV):
        hs.append(pltpu.async_copy(
            wloc.at[pl.ds(d * TPW, TPW)], w_hbm.at[d, pl.ds(t0, TPW)], wsem))
    for h in hs:
        h.wait()


def _routing_mask(idx_flat, expert_mapping):
    mesh = plsc.VectorSubcoreMesh(
        core_axis_name="c", subcore_axis_name="s",
        num_cores=NC, num_subcores=NS)
    f = pl.kernel(
        _routing_body,
        out_type=jax.ShapeDtypeStruct((NDEV, T), jnp.float32),
        mesh=mesh,
        compiler_params=pltpu.CompilerParams(needs_layout_passes=False),
        scratch_types=[
            pltpu.VMEM((TPW * TOP_K,), jnp.int32),
            pltpu.VMEM((NEXP,), jnp.int32),
            pltpu.VMEM((NDEV * TPW,), jnp.float32),
            pltpu.SemaphoreType.DMA,
        ],
    )
    return f(idx_flat, expert_mapping)


def _dispatch_tc(in_ref, w_ref, out_ref):
    for tc in range(TB // 8):
        rows8 = in_ref[pl.ds(tc * 8, 8), :]
        wb8 = w_ref[:, pl.ds(tc * 8, 8)]
        out_ref[:, pl.ds(tc * 8, 8), :] = rows8[None, :, :] * wb8[:, :, None]


def kernel(input_tensor, expert_indices, expert_mapping):
    idx_flat = expert_indices.reshape(-1)
    w = _routing_mask(idx_flat, expert_mapping)
    out = pl.pallas_call(
        _dispatch_tc,
        grid=(NTB,),
        in_specs=[
            pl.BlockSpec((TB, D), lambda tb: (tb, 0)),
            pl.BlockSpec((NDEV, TB), lambda tb: (0, tb)),
        ],
        out_specs=pl.BlockSpec((NDEV, TB, D), lambda tb: (0, tb, 0)),
        out_shape=jax.ShapeDtypeStruct((NDEV, T, D), jnp.float32),
    )(input_tensor, w)
    return out
